# Initial kernel scaffold; baseline (speedup 1.0000x reference)
import jax
import jax.numpy as jnp
from jax.experimental import pallas as pl


def kernel(x, router_W, fc1_w, fc1_b, fc2_w, fc2_b, is_training):
    raise NotImplementedError("write your pallas kernel here")


# speedup vs baseline: 2.0321x; 2.0321x over previous
import functools

import jax
import jax.numpy as jnp
from jax.experimental import pallas as pl
from jax.experimental.pallas import tpu as pltpu

B, S, D, H, E, K = 1, 2048, 1024, 2048, 8, 2
N = B * S
BM = 256
NB = N // BM


def _router_body(x_ref, rw_ref, w_ref, stats_ref, loss_ref):
    nb = pl.program_id(0)
    xb = x_ref[...]
    logits = jax.lax.dot_general(
        xb, rw_ref[...], (((1,), (1,)), ((), ())),
        preferred_element_type=jnp.float32)
    m = jnp.max(logits, axis=-1, keepdims=True)
    p = jnp.exp(logits - m)
    probs = p / jnp.sum(p, axis=-1, keepdims=True)

    lane = jax.lax.broadcasted_iota(jnp.int32, (BM, E), 1)
    p1 = jnp.max(probs, axis=-1, keepdims=True)
    i1 = jnp.min(jnp.where(probs == p1, lane, E), axis=-1, keepdims=True)
    m1 = lane == i1
    probs2 = jnp.where(m1, -jnp.inf, probs)
    p2 = jnp.max(probs2, axis=-1, keepdims=True)
    i2 = jnp.min(jnp.where(probs2 == p2, lane, E), axis=-1, keepdims=True)
    m2 = lane == i2

    denom = p1 + p2 + 1e-8
    w = jnp.where(m1, p1 / denom, 0.0) + jnp.where(m2, p2 / denom, 0.0)
    w_ref[...] = w

    psum = jnp.sum(probs, axis=0, keepdims=True)
    csum = jnp.sum((m1 | m2).astype(jnp.float32), axis=0, keepdims=True)
    contrib = jnp.concatenate([psum, csum], axis=0)

    @pl.when(nb == 0)
    def _():
        stats_ref[...] = jnp.zeros_like(stats_ref)

    stats_ref[...] += contrib

    @pl.when(nb == NB - 1)
    def _():
        st = stats_ref[...]
        mean_probs = st[0:1, :] / N
        fracs = st[1:2, :] / (N * K)
        loss_ref[...] = jnp.sum(mean_probs * fracs, keepdims=True).reshape(1, 1) * E


def _router(x_flat, router_W):
    return pl.pallas_call(
        _router_body,
        grid=(NB,),
        in_specs=[
            pl.BlockSpec((BM, D), lambda nb: (nb, 0)),
            pl.BlockSpec((E, D), lambda nb: (0, 0)),
        ],
        out_specs=[
            pl.BlockSpec((BM, E), lambda nb: (nb, 0)),
            pl.BlockSpec((2, E), lambda nb: (0, 0)),
            pl.BlockSpec((1, 1), lambda nb: (0, 0)),
        ],
        out_shape=[
            jax.ShapeDtypeStruct((N, E), jnp.float32),
            jax.ShapeDtypeStruct((2, E), jnp.float32),
            jax.ShapeDtypeStruct((1, 1), jnp.float32),
        ],
    )(x_flat, router_W)


def _ffn_body(x_ref, w1_ref, b1_ref, w2_ref, b2_ref, wcol_ref, out_ref, acc_ref):
    e = pl.program_id(0)
    nb = pl.program_id(1)
    xb = x_ref[...]
    h = jax.lax.dot_general(
        xb, w1_ref[0], (((1,), (1,)), ((), ())),
        preferred_element_type=jnp.float32) + b1_ref[0]
    h = 0.5 * h * (1.0 + jax.lax.erf(h * 0.7071067811865476))
    eo = jax.lax.dot_general(
        h, w2_ref[0], (((1,), (1,)), ((), ())),
        preferred_element_type=jnp.float32) + b2_ref[0]
    weighted = eo * wcol_ref[0]

    sl = (pl.ds(nb * BM, BM), slice(None))

    @pl.when(e == 0)
    def _():
        acc_ref[sl] = weighted

    @pl.when(e > 0)
    def _():
        acc_ref[sl] += weighted

    @pl.when(e == E - 1)
    def _():
        out_ref[...] = acc_ref[sl]


def _ffn(x_flat, fc1_w, fc1_b, fc2_w, fc2_b, wcols):
    return pl.pallas_call(
        _ffn_body,
        grid=(E, NB),
        in_specs=[
            pl.BlockSpec((BM, D), lambda e, nb: (nb, 0)),
            pl.BlockSpec((1, H, D), lambda e, nb: (e, 0, 0)),
            pl.BlockSpec((1, 1, H), lambda e, nb: (e, 0, 0)),
            pl.BlockSpec((1, D, H), lambda e, nb: (e, 0, 0)),
            pl.BlockSpec((1, 1, D), lambda e, nb: (e, 0, 0)),
            pl.BlockSpec((1, BM, 1), lambda e, nb: (e, nb, 0)),
        ],
        out_specs=pl.BlockSpec((BM, D), lambda e, nb: (nb, 0)),
        out_shape=jax.ShapeDtypeStruct((N, D), jnp.float32),
        scratch_shapes=[pltpu.VMEM((N, D), jnp.float32)],
    )(x_flat, fc1_w, fc1_b.reshape(E, 1, H), fc2_w, fc2_b.reshape(E, 1, D), wcols)


def kernel(x, router_W, fc1_w, fc1_b, fc2_w, fc2_b, is_training):
    x_flat = x.reshape(N, D)
    w_full, _stats, loss = _router(x_flat, router_W)
    wcols = w_full.T.reshape(E, N, 1)
    out = _ffn(x_flat, fc1_w, fc1_b, fc2_w, fc2_b, wcols)
    return out.reshape(x.shape), loss.reshape(())
